# baseline (speedup 1.0000x reference)
import functools

import jax
import jax.numpy as jnp
from jax.experimental import pallas as pl
from jax.experimental.pallas import tpu as pltpu

_E = 64
_K = 8
_ALPHA = 0.01
_MASKED = -1e30


def _router_kernel(hs_ref, wt_ref, idx_ref, w_ref, aux_ref,
                   cnt_acc, ssum_acc, *, nb, bpb, s_len, b_sz):
    i = pl.program_id(0)
    hs = hs_ref[...]
    logits = jnp.dot(hs, wt_ref[...], preferred_element_type=jnp.float32)

    ex = jnp.exp(logits)
    zinv = 1.0 / jnp.sum(ex, axis=-1, keepdims=True)
    blk_ssum = jnp.sum(ex * zinv, axis=0, keepdims=True)

    r = logits.shape[0]
    col8 = jax.lax.broadcasted_iota(jnp.int32, (r, _K), 1)
    erow = jax.lax.broadcasted_iota(jnp.int32, (_E, _K), 0)
    kcol = jax.lax.broadcasted_iota(jnp.int32, (_E, _K), 1)
    work = logits
    idx_mat = jnp.zeros((r, _K), jnp.float32)
    val_mat = jnp.zeros((r, _K), jnp.float32)
    for k in range(_K):
        mx = jnp.max(work, axis=-1, keepdims=True)
        eq = work == mx
        ef = jnp.where(eq, 1.0, 0.0)
        ck = jnp.where(kcol == k, erow, 0).astype(jnp.float32)
        idx_mat = idx_mat + jnp.dot(ef, ck, preferred_element_type=jnp.float32)
        val_mat = jnp.where(col8 == k, mx, val_mat)
        work = jnp.where(eq, _MASKED, work)
    blk_cnt = jnp.sum(jnp.where(work <= _MASKED, 1.0, 0.0), axis=0, keepdims=True)
    ex8 = jnp.exp(val_mat)
    denom = jnp.sum(ex8, axis=-1, keepdims=True)
    idx_ref[...] = idx_mat.astype(jnp.int32)
    w_ref[...] = ex8 / denom

    b = i // bpb

    @pl.when(i % bpb == 0)
    def _():
        cnt_acc[pl.ds(b, 1), :] = blk_cnt
        ssum_acc[pl.ds(b, 1), :] = blk_ssum

    @pl.when(i % bpb != 0)
    def _():
        cnt_acc[pl.ds(b, 1), :] += blk_cnt
        ssum_acc[pl.ds(b, 1), :] += blk_ssum

    @pl.when(i == nb - 1)
    def _():
        ce = cnt_acc[...] * (_E / (s_len * _K))
        ms = ssum_acc[...] / s_len
        aux_ref[...] = jnp.sum(ce * ms, keepdims=True).reshape(1, 1) * (_ALPHA / b_sz)


def kernel(hidden_states, weight):
    b, s, d = hidden_states.shape
    n = b * s
    hs = hidden_states.reshape(n, d)
    wt = weight.T
    blk = 1024
    nb = n // blk
    bpb = s // blk

    idx, w, aux = pl.pallas_call(
        functools.partial(_router_kernel, nb=nb, bpb=bpb, s_len=s, b_sz=b),
        grid=(nb,),
        in_specs=[
            pl.BlockSpec((blk, d), lambda i: (i, 0)),
            pl.BlockSpec((d, _E), lambda i: (0, 0)),
        ],
        out_specs=[
            pl.BlockSpec((blk, _K), lambda i: (i, 0)),
            pl.BlockSpec((blk, _K), lambda i: (i, 0)),
            pl.BlockSpec((1, 1), lambda i: (0, 0)),
        ],
        out_shape=[
            jax.ShapeDtypeStruct((n, _K), jnp.int32),
            jax.ShapeDtypeStruct((n, _K), jnp.float32),
            jax.ShapeDtypeStruct((1, 1), jnp.float32),
        ],
        scratch_shapes=[
            pltpu.VMEM((b, _E), jnp.float32),
            pltpu.VMEM((b, _E), jnp.float32),
        ],
        compiler_params=pltpu.CompilerParams(
            dimension_semantics=("arbitrary",),
        ),
    )(hs, wt)
    return idx, w, aux[0, 0]


# speedup vs baseline: 1.0193x; 1.0193x over previous
import functools

import jax
import jax.numpy as jnp
from jax.experimental import pallas as pl
from jax.experimental.pallas import tpu as pltpu

_E = 64
_K = 8
_ALPHA = 0.01
_MASKED = -1e30


def _router_kernel(hs_ref, wt_ref, idx_ref, w_ref, cnt_ref, ssum_ref, *, bpb):
    i = pl.program_id(0)
    hs = hs_ref[...]
    logits = jnp.dot(hs, wt_ref[...], preferred_element_type=jnp.float32)

    ex = jnp.exp(logits)
    zinv = 1.0 / jnp.sum(ex, axis=-1, keepdims=True)
    blk_ssum = jnp.sum(ex * zinv, axis=0, keepdims=True)

    r = logits.shape[0]
    col8 = jax.lax.broadcasted_iota(jnp.int32, (r, _K), 1)
    erow = jax.lax.broadcasted_iota(jnp.int32, (_E, _K), 0)
    kcol = jax.lax.broadcasted_iota(jnp.int32, (_E, _K), 1)
    work = logits
    idx_mat = jnp.zeros((r, _K), jnp.float32)
    val_mat = jnp.zeros((r, _K), jnp.float32)
    for k in range(_K):
        mx = jnp.max(work, axis=-1, keepdims=True)
        eq = work == mx
        ef = jnp.where(eq, 1.0, 0.0)
        ck = jnp.where(kcol == k, erow, 0).astype(jnp.float32)
        idx_mat = idx_mat + jnp.dot(ef, ck, preferred_element_type=jnp.float32)
        val_mat = jnp.where(col8 == k, mx, val_mat)
        work = jnp.where(eq, _MASKED, work)
    blk_cnt = jnp.sum(jnp.where(work <= _MASKED, 1.0, 0.0), axis=0, keepdims=True)
    ex8 = jnp.exp(val_mat)
    denom = jnp.sum(ex8, axis=-1, keepdims=True)
    idx_ref[...] = idx_mat.astype(jnp.int32)
    w_ref[...] = ex8 / denom

    @pl.when(i % bpb == 0)
    def _():
        cnt_ref[0, :, :] = blk_cnt
        ssum_ref[0, :, :] = blk_ssum

    @pl.when(i % bpb != 0)
    def _():
        cnt_ref[0, :, :] += blk_cnt
        ssum_ref[0, :, :] += blk_ssum


def _aux_kernel(cnt_ref, ssum_ref, aux_ref, *, s_len, b_sz):
    ce = cnt_ref[...] * (_E / (s_len * _K))
    ms = ssum_ref[...] / s_len
    aux_ref[...] = jnp.sum(ce * ms, keepdims=True).reshape(1, 1) * (_ALPHA / b_sz)


def kernel(hidden_states, weight):
    b, s, d = hidden_states.shape
    n = b * s
    hs = hidden_states.reshape(n, d)
    wt = weight.T
    blk = 2048
    nb = n // blk
    bpb = s // blk

    idx, w, cnt, ssum = pl.pallas_call(
        functools.partial(_router_kernel, bpb=bpb),
        grid=(nb,),
        in_specs=[
            pl.BlockSpec((blk, d), lambda i: (i, 0)),
            pl.BlockSpec((d, _E), lambda i: (0, 0)),
        ],
        out_specs=[
            pl.BlockSpec((blk, _K), lambda i: (i, 0)),
            pl.BlockSpec((blk, _K), lambda i: (i, 0)),
            pl.BlockSpec((1, 1, _E), lambda i: (i // bpb, 0, 0)),
            pl.BlockSpec((1, 1, _E), lambda i: (i // bpb, 0, 0)),
        ],
        out_shape=[
            jax.ShapeDtypeStruct((n, _K), jnp.int32),
            jax.ShapeDtypeStruct((n, _K), jnp.float32),
            jax.ShapeDtypeStruct((b, 1, _E), jnp.float32),
            jax.ShapeDtypeStruct((b, 1, _E), jnp.float32),
        ],
        compiler_params=pltpu.CompilerParams(
            dimension_semantics=("arbitrary",),
        ),
    )(hs, wt)

    aux = pl.pallas_call(
        functools.partial(_aux_kernel, s_len=s, b_sz=b),
        out_shape=jax.ShapeDtypeStruct((1, 1), jnp.float32),
    )(cnt, ssum)
    return idx, w, aux[0, 0]


# speedup vs baseline: 1.0409x; 1.0212x over previous
import functools

import jax
import jax.numpy as jnp
from jax.experimental import pallas as pl
from jax.experimental.pallas import tpu as pltpu

_E = 64
_K = 8
_ALPHA = 0.01
_MASKED = -1e30


def _router_kernel(hs_ref, wt_ref, idx_ref, w_ref, cnt_ref, ssum_ref,
                   aux_ref, *, nb, bpb, s_len, b_sz):
    i = pl.program_id(0)
    hs = hs_ref[...]
    logits = jnp.dot(hs, wt_ref[...], preferred_element_type=jnp.float32)

    ex = jnp.exp(logits)
    zinv = 1.0 / jnp.sum(ex, axis=-1, keepdims=True)
    blk_ssum = jnp.sum(ex * zinv, axis=0, keepdims=True)

    r = logits.shape[0]
    col8 = jax.lax.broadcasted_iota(jnp.int32, (r, _K), 1)
    erow = jax.lax.broadcasted_iota(jnp.int32, (_E, _K), 0)
    kcol = jax.lax.broadcasted_iota(jnp.int32, (_E, _K), 1)
    work = logits
    idx_mat = jnp.zeros((r, _K), jnp.float32)
    val_mat = jnp.zeros((r, _K), jnp.float32)
    for k in range(_K):
        mx = jnp.max(work, axis=-1, keepdims=True)
        eq = work == mx
        ef = jnp.where(eq, 1.0, 0.0)
        ck = jnp.where(kcol == k, erow, 0).astype(jnp.float32)
        idx_mat = idx_mat + jnp.dot(ef, ck, preferred_element_type=jnp.float32)
        val_mat = jnp.where(col8 == k, mx, val_mat)
        work = jnp.where(eq, _MASKED, work)
    blk_cnt = jnp.sum(jnp.where(work <= _MASKED, 1.0, 0.0), axis=0, keepdims=True)
    ex8 = jnp.exp(val_mat)
    denom = jnp.sum(ex8, axis=-1, keepdims=True)
    idx_ref[...] = idx_mat.astype(jnp.int32)
    w_ref[...] = ex8 / denom

    brow = jax.lax.broadcasted_iota(jnp.int32, (8, _E), 0)
    inrow = brow == i // bpb
    cnt_add = jnp.where(inrow, blk_cnt, 0.0)
    ssum_add = jnp.where(inrow, blk_ssum, 0.0)

    @pl.when(i == 0)
    def _():
        cnt_ref[0, :, :] = cnt_add
        ssum_ref[0, :, :] = ssum_add

    @pl.when(i != 0)
    def _():
        cnt_ref[0, :, :] += cnt_add
        ssum_ref[0, :, :] += ssum_add

    @pl.when(i == nb - 1)
    def _():
        ce = cnt_ref[0, :, :] * (_E / (s_len * _K))
        ms = ssum_ref[0, :, :] / s_len
        aux_ref[...] = jnp.sum(ce * ms, keepdims=True).reshape(1, 1) * (_ALPHA / b_sz)


def kernel(hidden_states, weight):
    b, s, d = hidden_states.shape
    n = b * s
    hs = hidden_states.reshape(n, d)
    wt = weight.T
    blk = 2048
    nb = n // blk
    bpb = s // blk

    idx, w, _, _, aux = pl.pallas_call(
        functools.partial(_router_kernel, nb=nb, bpb=bpb, s_len=s, b_sz=b),
        grid=(nb,),
        in_specs=[
            pl.BlockSpec((blk, d), lambda i: (i, 0)),
            pl.BlockSpec((d, _E), lambda i: (0, 0)),
        ],
        out_specs=[
            pl.BlockSpec((blk, _K), lambda i: (i, 0)),
            pl.BlockSpec((blk, _K), lambda i: (i, 0)),
            pl.BlockSpec((1, 8, _E), lambda i: (0, 0, 0)),
            pl.BlockSpec((1, 8, _E), lambda i: (0, 0, 0)),
            pl.BlockSpec((1, 1), lambda i: (0, 0)),
        ],
        out_shape=[
            jax.ShapeDtypeStruct((n, _K), jnp.int32),
            jax.ShapeDtypeStruct((n, _K), jnp.float32),
            jax.ShapeDtypeStruct((1, 8, _E), jnp.float32),
            jax.ShapeDtypeStruct((1, 8, _E), jnp.float32),
            jax.ShapeDtypeStruct((1, 1), jnp.float32),
        ],
        compiler_params=pltpu.CompilerParams(
            dimension_semantics=("arbitrary",),
        ),
    )(hs, wt)
    return idx, w, aux[0, 0]
